# Initial kernel scaffold; baseline (speedup 1.0000x reference)
import jax
import jax.numpy as jnp
from jax.experimental import pallas as pl


def kernel(h, e, edge_index, A_w, A_b, B_w, B_b, C_w, C_b, D_w, D_b, E_w, E_b, bn_h_w, bn_h_b, bn_h_mean, bn_h_var, bn_e_w, bn_e_b, bn_e_mean, bn_e_var):
    raise NotImplementedError("write your pallas kernel here")


# speedup vs baseline: 1.4684x; 1.4684x over previous
import functools

import jax
import jax.numpy as jnp
from jax import lax
from jax.experimental import pallas as pl
from jax.experimental.pallas import tpu as pltpu
from jax.experimental.pallas import tpu_sc as plsc

_N = 10000
_E = 320000
_D = 128
_H = 64
_C = 64
_NCHUNK = _E // _C
_NSUB = 16
_GSTEPS = (_NCHUNK + _NSUB - 1) // _NSUB
_ROWS_PER_SUB = _N // _NSUB

_f32 = jnp.float32


def _proj_body(h_ref, w_ref, b_ref, o_ref):
    o_ref[...] = (
        jnp.dot(h_ref[...], w_ref[...], preferred_element_type=_f32)
        + b_ref[0:1, :]
    )


def _proj_call(h, wcat, bcat):
    bn = 2000
    grid = (_N // bn,)
    return pl.pallas_call(
        _proj_body,
        grid=grid,
        in_specs=[
            pl.BlockSpec((bn, _D), lambda i: (i, 0)),
            pl.BlockSpec((_D, 512), lambda i: (0, 0)),
            pl.BlockSpec((8, 512), lambda i: (0, 0)),
        ],
        out_specs=pl.BlockSpec((bn, 512), lambda i: (i, 0)),
        out_shape=jax.ShapeDtypeStruct((_N, 512), _f32),
    )(h, wcat, bcat)


def _ce_body(e_ref, w_ref, b_ref, o_ref):
    o_ref[...] = (
        jnp.dot(e_ref[...], w_ref[...], preferred_element_type=_f32)
        + b_ref[0:1, :]
    )


def _ce_call(e, wt, b):
    be = 3200
    grid = (_E // be,)
    return pl.pallas_call(
        _ce_body,
        grid=grid,
        in_specs=[
            pl.BlockSpec((be, _D), lambda i: (i, 0)),
            pl.BlockSpec((_D, _D), lambda i: (0, 0)),
            pl.BlockSpec((8, _D), lambda i: (0, 0)),
        ],
        out_specs=pl.BlockSpec((be, _D), lambda i: (i, 0)),
        out_shape=jax.ShapeDtypeStruct((_E, _D), _f32),
    )(e, wt, b)


def _edge_body(tsrc_hbm, tdst_hbm, ce_hbm, ei_hbm, sb_hbm,
               eout_hbm, num_hbm, den_hbm,
               srcoff_v, dst_v, dstoff_v, sb_v,
               tsrc_b, tdst_b, ce_b, ms_b,
               acc, sem_a, sem_b):
    c = lax.axis_index("c")
    s = lax.axis_index("s")

    pltpu.sync_copy(sb_hbm.at[c], sb_v)

    def _zfill(r, carry):
        z = jnp.zeros((16,), _f32)
        for j in range(_D // 16):
            ms_b[r, pl.ds(j * 16, 16)] = z
        return carry
    lax.fori_loop(0, _C, _zfill, None)
    row0 = s * _ROWS_PER_SUB
    for k in range(_ROWS_PER_SUB // _C):
        pltpu.sync_copy(ms_b.at[pl.ds(0, _C)],
                        acc.at[pl.ds(row0 + k * _C, _C)])
    _rem = _ROWS_PER_SUB % _C
    if _rem:
        pltpu.sync_copy(ms_b.at[pl.ds(0, _rem)],
                        acc.at[pl.ds(row0 + (_ROWS_PER_SUB // _C) * _C, _rem)])
    plsc.subcore_barrier()

    off_vec = jnp.broadcast_to((c * _N).astype(jnp.int32), (16,))

    scs = [sb_v[0, pl.ds(j * 16, 16)] for j in range(_H // 16)]
    bis = [sb_v[1, pl.ds(j * 16, 16)] for j in range(_H // 16)]

    def _chunk(g, carry):
        cid = g * _NSUB + s

        @pl.when(cid < _NCHUNK)
        def _():
            base = cid * _C
            pltpu.sync_copy(ei_hbm.at[0, pl.ds(base, _C)], srcoff_v)
            pltpu.sync_copy(ei_hbm.at[1, pl.ds(base, _C)], dst_v)

            def _off(i, carry2):
                sl = pl.ds(i * 16, 16)
                srcoff_v[sl] = srcoff_v[sl] + off_vec
                dstoff_v[sl] = dst_v[sl] + off_vec
                return carry2
            lax.fori_loop(0, _C // 16, _off, None)

            cp1 = pltpu.async_copy(tsrc_hbm.at[srcoff_v], tsrc_b, sem_a)
            cp2 = pltpu.async_copy(tdst_hbm.at[dstoff_v], tdst_b, sem_b)

            @pl.when(c == 0)
            def _():
                pltpu.sync_copy(ce_hbm.at[pl.ds(base, _C), pl.ds(0, _H)], ce_b)

            @pl.when(c == 1)
            def _():
                pltpu.sync_copy(ce_hbm.at[pl.ds(base, _C), pl.ds(_H, _H)], ce_b)

            cp1.wait()
            cp2.wait()

            def _row(r, carry2):
                for j in range(_H // 16):
                    sl = pl.ds(j * 16, 16)
                    eij = ce_b[r, sl] + tsrc_b[r, sl] + tdst_b[r, sl]
                    sg = 1.0 / (1.0 + jnp.exp(-eij))
                    ce_b[r, sl] = jnp.maximum(eij * scs[j] + bis[j], 0.0)
                    ms_b[r, sl] = sg * tsrc_b[r, pl.ds(_H + j * 16, 16)]
                    ms_b[r, pl.ds(_H + j * 16, 16)] = sg
                return carry2
            lax.fori_loop(0, _C, _row, None)

            pltpu.sync_copy(ms_b, acc.at[dst_v], add=True)

            @pl.when(c == 0)
            def _():
                pltpu.sync_copy(ce_b, eout_hbm.at[pl.ds(base, _C), pl.ds(0, _H)])

            @pl.when(c == 1)
            def _():
                pltpu.sync_copy(ce_b, eout_hbm.at[pl.ds(base, _C), pl.ds(_H, _H)])
        return carry
    lax.fori_loop(0, _GSTEPS, _chunk, None)

    plsc.subcore_barrier()

    rows = pl.ds(row0, _ROWS_PER_SUB)

    @pl.when(c == 0)
    def _():
        pltpu.sync_copy(acc.at[rows, pl.ds(0, _H)], num_hbm.at[rows, pl.ds(0, _H)])
        pltpu.sync_copy(acc.at[rows, pl.ds(_H, _H)], den_hbm.at[rows, pl.ds(0, _H)])

    @pl.when(c == 1)
    def _():
        pltpu.sync_copy(acc.at[rows, pl.ds(0, _H)], num_hbm.at[rows, pl.ds(_H, _H)])
        pltpu.sync_copy(acc.at[rows, pl.ds(_H, _H)], den_hbm.at[rows, pl.ds(_H, _H)])


def _edge_call(tsrc, tdst, ce, ei, sb):
    mesh = plsc.VectorSubcoreMesh(core_axis_name="c", subcore_axis_name="s")
    f = functools.partial(
        pl.kernel,
        mesh=mesh,
        compiler_params=pltpu.CompilerParams(use_tc_tiling_on_sc=False),
        out_type=[
            jax.ShapeDtypeStruct((_E, _D), _f32),
            jax.ShapeDtypeStruct((_N, _D), _f32),
            jax.ShapeDtypeStruct((_N, _D), _f32),
        ],
        scratch_types=[
            pltpu.VMEM((_C,), jnp.int32),
            pltpu.VMEM((_C,), jnp.int32),
            pltpu.VMEM((_C,), jnp.int32),
            pltpu.VMEM((2, _H), _f32),
            pltpu.VMEM((_C, _D), _f32),
            pltpu.VMEM((_C, _H), _f32),
            pltpu.VMEM((_C, _H), _f32),
            pltpu.VMEM((_C, _D), _f32),
            pltpu.VMEM_SHARED((_N, _D), _f32),
            pltpu.SemaphoreType.DMA,
            pltpu.SemaphoreType.DMA,
        ],
    )(_edge_body)
    return f(tsrc, tdst, ce, ei, sb)


def _final_body(ah_ref, num_ref, den_ref, sc_ref, bi_ref, o_ref):
    hn = ah_ref[...] + num_ref[...] / (den_ref[...] + 1e-6)
    o_ref[...] = jnp.maximum(hn * sc_ref[0:1, :] + bi_ref[0:1, :], 0.0)


def _final_call(ah, num, den, scale, bias):
    bn = 2000
    grid = (_N // bn,)
    return pl.pallas_call(
        _final_body,
        grid=grid,
        in_specs=[
            pl.BlockSpec((bn, _D), lambda i: (i, 0)),
            pl.BlockSpec((bn, _D), lambda i: (i, 0)),
            pl.BlockSpec((bn, _D), lambda i: (i, 0)),
            pl.BlockSpec((8, _D), lambda i: (0, 0)),
            pl.BlockSpec((8, _D), lambda i: (0, 0)),
        ],
        out_specs=pl.BlockSpec((bn, _D), lambda i: (i, 0)),
        out_shape=jax.ShapeDtypeStruct((_N, _D), _f32),
    )(ah, num, den, scale, bias)


def kernel(h, e, edge_index, A_w, A_b, B_w, B_b, C_w, C_b, D_w, D_b, E_w, E_b,
           bn_h_w, bn_h_b, bn_h_mean, bn_h_var, bn_e_w, bn_e_b, bn_e_mean,
           bn_e_var):
    wcat = jnp.concatenate([D_w.T, B_w.T, E_w.T, A_w.T], axis=1)
    bcat = jnp.broadcast_to(
        jnp.concatenate([D_b, B_b, E_b, A_b])[None, :], (8, 512))

    proj = _proj_call(h, wcat, bcat)
    dh = proj[:, 0:128]
    bh = proj[:, 128:256]
    eh = proj[:, 256:384]
    ah = proj[:, 384:512]

    tsrc = jnp.concatenate(
        [jnp.concatenate([dh[:, :_H], bh[:, :_H]], axis=1),
         jnp.concatenate([dh[:, _H:], bh[:, _H:]], axis=1)], axis=0)
    tdst = jnp.concatenate([eh[:, :_H], eh[:, _H:]], axis=0)

    ce = _ce_call(e, C_w.T, jnp.broadcast_to(C_b[None, :], (8, _D)))

    sc_h = bn_h_w / jnp.sqrt(bn_h_var + 1e-5)
    bi_h = bn_h_b - bn_h_mean * sc_h
    sc_e = bn_e_w / jnp.sqrt(bn_e_var + 1e-5)
    bi_e = bn_e_b - bn_e_mean * sc_e
    sb = jnp.stack([jnp.stack([sc_e[:_H], bi_e[:_H]]),
                    jnp.stack([sc_e[_H:], bi_e[_H:]])])

    ei = edge_index.astype(jnp.int32)

    e_out, num, den = _edge_call(tsrc, tdst, ce, ei, sb)

    h_out = _final_call(ah, num, den,
                        jnp.broadcast_to(sc_h[None, :], (8, _D)),
                        jnp.broadcast_to(bi_h[None, :], (8, _D)))
    return (h_out, e_out)
